# baseline (speedup 1.0000x reference)
import functools

import jax
import jax.numpy as jnp
from jax import lax
from jax.experimental import pallas as pl
from jax.experimental.pallas import tpu as pltpu
from jax.experimental.pallas import tpu_sc as plsc

_B = 16384
_D = 512
_OUT_LO = 120
_OUT_HI = 136
_NC = 2
_NS = 16
_NW = _NC * _NS
_RPW = _B // _NW
_C = 64
_NCHUNK = _RPW // _C


def _sc_body(x_hbm, out_hbm, buf):
    wid = lax.axis_index("s") * _NC + lax.axis_index("c")
    base = wid * _RPW
    lane = lax.iota(jnp.int32, 16)
    hit_lo = lane == (_OUT_LO % 16)
    hit_hi = lane == (_OUT_HI % 16)
    two = jnp.full((16,), 2.0, jnp.float32)
    lo_base = (_OUT_LO // 16) * 16
    hi_base = (_OUT_HI // 16) * 16
    for i in range(_NCHUNK):
        pltpu.sync_copy(x_hbm.at[pl.ds(base + i * _C, _C)], buf)
        for r in range(_C):
            buf[r, pl.ds(lo_base, 16)] = jnp.where(
                hit_lo, two, buf[r, pl.ds(lo_base, 16)])
            buf[r, pl.ds(hi_base, 16)] = jnp.where(
                hit_hi, two, buf[r, pl.ds(hi_base, 16)])
        pltpu.sync_copy(buf, out_hbm.at[pl.ds(base + i * _C, _C)])


_sc_call = functools.partial(
    pl.kernel,
    out_type=jax.ShapeDtypeStruct((_B, _D), jnp.float32),
    mesh=plsc.VectorSubcoreMesh(core_axis_name="c", subcore_axis_name="s"),
    scratch_types=[pltpu.VMEM((_C, _D), jnp.float32)],
    compiler_params=pltpu.CompilerParams(
        needs_layout_passes=False, use_tc_tiling_on_sc=True),
)(_sc_body)


def kernel(x_bd):
    out = _sc_call(x_bd.reshape(_B, _D))
    return out.reshape(_B, 1, _D)


# speedup vs baseline: 2.0889x; 2.0889x over previous
import functools

import jax
import jax.numpy as jnp
from jax import lax
from jax.experimental import pallas as pl
from jax.experimental.pallas import tpu as pltpu
from jax.experimental.pallas import tpu_sc as plsc

_B = 16384
_D = 512
_OUT_LO = 120
_OUT_HI = 136
_NC = 2
_NS = 16
_NW = _NC * _NS
_RPW = _B // _NW
_C = 64
_NCHUNK = _RPW // _C


def _sc_body(x_hbm, out_hbm, buf):
    wid = lax.axis_index("s") * _NC + lax.axis_index("c")
    base = wid * _RPW
    lane = lax.iota(jnp.int32, 16)
    hit_lo = lane == (_OUT_LO % 16)
    hit_hi = lane == (_OUT_HI % 16)
    two = jnp.full((16,), 2.0, jnp.float32)
    lo_base = (_OUT_LO // 16) * 16
    hi_base = (_OUT_HI // 16) * 16
    for i in range(_NCHUNK):
        pltpu.sync_copy(x_hbm.at[pl.ds(base + i * _C, _C)], buf)
        for r in range(_C):
            buf[r, pl.ds(lo_base, 16)] = jnp.where(
                hit_lo, two, buf[r, pl.ds(lo_base, 16)])
            buf[r, pl.ds(hi_base, 16)] = jnp.where(
                hit_hi, two, buf[r, pl.ds(hi_base, 16)])
        pltpu.sync_copy(buf, out_hbm.at[pl.ds(base + i * _C, _C)])


_sc_call = functools.partial(
    pl.kernel,
    out_type=jax.ShapeDtypeStruct((_B, _D), jnp.float32),
    mesh=plsc.VectorSubcoreMesh(core_axis_name="c", subcore_axis_name="s"),
    scratch_types=[pltpu.VMEM((_C, _D), jnp.float32)],
    compiler_params=pltpu.CompilerParams(
        needs_layout_passes=False, use_tc_tiling_on_sc=False),
)(_sc_body)


def kernel(x_bd):
    out = _sc_call(x_bd.reshape(_B, _D))
    return out.reshape(_B, 1, _D)


# speedup vs baseline: 2.3228x; 1.1120x over previous
import functools

import jax
import jax.numpy as jnp
from jax import lax
from jax.experimental import pallas as pl
from jax.experimental.pallas import tpu as pltpu
from jax.experimental.pallas import tpu_sc as plsc

_B = 16384
_D = 512
_OUT_LO = 120
_OUT_HI = 136
_NC = 2
_NS = 16
_NW = _NC * _NS
_RPW = _B // _NW
_C = 64
_NCHUNK = _RPW // _C


def _overwrite(buf, hit_lo, hit_hi, two, lo_base, hi_base):
    for r in range(_C):
        buf[r, pl.ds(lo_base, 16)] = jnp.where(
            hit_lo, two, buf[r, pl.ds(lo_base, 16)])
        buf[r, pl.ds(hi_base, 16)] = jnp.where(
            hit_hi, two, buf[r, pl.ds(hi_base, 16)])


def _sc_body(x_hbm, out_hbm, buf0, buf1, si0, si1, so0, so1):
    wid = lax.axis_index("s") * _NC + lax.axis_index("c")
    base = wid * _RPW
    lane = lax.iota(jnp.int32, 16)
    hit_lo = lane == (_OUT_LO % 16)
    hit_hi = lane == (_OUT_HI % 16)
    two = jnp.full((16,), 2.0, jnp.float32)
    lo_base = (_OUT_LO // 16) * 16
    hi_base = (_OUT_HI // 16) * 16
    bufs = (buf0, buf1)
    sin = (si0, si1)
    sout = (so0, so1)
    in_cp = [None, None]
    out_cp = [None, None]
    in_cp[0] = pltpu.async_copy(x_hbm.at[pl.ds(base, _C)], bufs[0], sin[0])
    for i in range(_NCHUNK):
        b = i % 2
        nb = 1 - b
        if i + 1 < _NCHUNK:
            if out_cp[nb] is not None:
                out_cp[nb].wait()
            in_cp[nb] = pltpu.async_copy(
                x_hbm.at[pl.ds(base + (i + 1) * _C, _C)], bufs[nb], sin[nb])
        in_cp[b].wait()
        _overwrite(bufs[b], hit_lo, hit_hi, two, lo_base, hi_base)
        out_cp[b] = pltpu.async_copy(
            bufs[b], out_hbm.at[pl.ds(base + i * _C, _C)], sout[b])
    out_cp[(_NCHUNK - 2) % 2].wait()
    out_cp[(_NCHUNK - 1) % 2].wait()


_sc_call = functools.partial(
    pl.kernel,
    out_type=jax.ShapeDtypeStruct((_B, _D), jnp.float32),
    mesh=plsc.VectorSubcoreMesh(core_axis_name="c", subcore_axis_name="s"),
    scratch_types=[
        pltpu.VMEM((_C, _D), jnp.float32),
        pltpu.VMEM((_C, _D), jnp.float32),
        pltpu.SemaphoreType.DMA,
        pltpu.SemaphoreType.DMA,
        pltpu.SemaphoreType.DMA,
        pltpu.SemaphoreType.DMA,
    ],
    compiler_params=pltpu.CompilerParams(
        needs_layout_passes=False, use_tc_tiling_on_sc=False),
)(_sc_body)


def kernel(x_bd):
    out = _sc_call(x_bd.reshape(_B, _D))
    return out.reshape(_B, 1, _D)
